# baseline (speedup 1.0000x reference)
import functools

import jax
import jax.numpy as jnp
from jax.experimental import pallas as pl
from jax.experimental.pallas import tpu as pltpu


def _fm_block(attrs_t_ref, emb_t_ref, w_t_ref, out_t_ref):
    out_t_ref[0:32, :] = emb_t_ref[...]
    out_t_ref[32:64, :] = jnp.dot(w_t_ref[...], attrs_t_ref[...],
                                  preferred_element_type=jnp.float32)


@functools.partial(jax.jit, static_argnames=("block_cols",))
def _fm(attrs, emb, w, block_cols):
    n, d_emb = emb.shape
    d_attr = attrs.shape[1]
    d_out = d_emb + w.shape[1]
    attrs_t = attrs.T
    emb_t = emb.T
    w_t = w.T
    grid = (pl.cdiv(n, block_cols),)
    out_t = pl.pallas_call(
        _fm_block,
        grid=grid,
        in_specs=[
            pl.BlockSpec((d_attr, block_cols), lambda i: (0, i)),
            pl.BlockSpec((d_emb, block_cols), lambda i: (0, i)),
            pl.BlockSpec((w.shape[1], d_attr), lambda i: (0, 0)),
        ],
        out_specs=pl.BlockSpec((d_out, block_cols), lambda i: (0, i)),
        compiler_params=pltpu.CompilerParams(
            dimension_semantics=("parallel",)),
        out_shape=jax.ShapeDtypeStruct((d_out, n), jnp.float32),
    )(attrs_t, emb_t, w_t)
    return out_t.T


def kernel(user_attrs_input, item_attrs_input, user_emb, item_emb,
           user_attrs_trans_w, item_attrs_trans_w):
    final_user = _fm(user_attrs_input, user_emb, user_attrs_trans_w, 65536)
    final_item = _fm(item_attrs_input, item_emb, item_attrs_trans_w, 65536)
    return (final_user, final_item)
